# Initial kernel scaffold; baseline (speedup 1.0000x reference)
import jax
import jax.numpy as jnp
from jax.experimental import pallas as pl


def kernel(hidden_states, gate_weight, e_bias, gate_proj, up_proj, down_proj, shared_gate_w, shared_up_w, shared_down_w):
    raise NotImplementedError("write your pallas kernel here")


# speedup vs baseline: 1.0704x; 1.0704x over previous
import functools

import jax
import jax.numpy as jnp
from jax.experimental import pallas as pl

H = 2048
I = 1408
E = 64
TOPK = 8
NG = 8
GS = E // NG
TG = 4
NSH = 2
SI = I * NSH
RSF = 2.5
T = 2048

NEG = -1e30


def _routing_kernel(x_ref, gw_ref, eb_ref, ws_ref):
    x = x_ref[...]
    gw = gw_ref[...]
    logits = jax.lax.dot_general(
        x, gw, (((1,), (1,)), ((), ())), preferred_element_type=jnp.float32
    )
    scores = jax.nn.sigmoid(logits)
    sfc = scores + eb_ref[...]

    gs_cols = []
    for g in range(NG):
        sl = sfc[:, g * GS:(g + 1) * GS]
        it = jax.lax.broadcasted_iota(jnp.int32, sl.shape, 1)
        m1 = jnp.max(sl, axis=1, keepdims=True)
        first = jnp.min(jnp.where(sl == m1, it, GS), axis=1, keepdims=True)
        m2 = jnp.max(jnp.where(it == first, NEG, sl), axis=1, keepdims=True)
        gs_cols.append(m1 + m2)
    gsc = jnp.concatenate(gs_cols, axis=1)

    itg = jax.lax.broadcasted_iota(jnp.int32, gsc.shape, 1)
    gmask = jnp.zeros_like(gsc)
    for _ in range(TG):
        m = jnp.max(gsc, axis=1, keepdims=True)
        first = jnp.min(jnp.where(gsc == m, itg, NG), axis=1, keepdims=True)
        sel = itg == first
        gmask = jnp.where(sel, 1.0, gmask)
        gsc = jnp.where(sel, NEG, gsc)
    smask = jnp.concatenate(
        [jnp.broadcast_to(gmask[:, g:g + 1], (gmask.shape[0], GS)) for g in range(NG)],
        axis=1,
    )

    tmp = jnp.where(smask > 0, sfc, 0.0)
    ite = jax.lax.broadcasted_iota(jnp.int32, tmp.shape, 1)
    wsel = jnp.zeros_like(tmp)
    denom = jnp.zeros((tmp.shape[0], 1), jnp.float32)
    for _ in range(TOPK):
        m = jnp.max(tmp, axis=1, keepdims=True)
        first = jnp.min(jnp.where(tmp == m, ite, E), axis=1, keepdims=True)
        sel = ite == first
        w = jnp.where(sel, scores, 0.0)
        wsel = wsel + w
        denom = denom + jnp.sum(w, axis=1, keepdims=True)
        tmp = jnp.where(sel, NEG, tmp)
    ws_ref[...] = wsel / (denom + 1e-20) * RSF


def _route(x, gate_weight, e_bias):
    return pl.pallas_call(
        _routing_kernel,
        out_shape=jax.ShapeDtypeStruct((T, E), jnp.float32),
    )(x, gate_weight, e_bias.reshape(1, E))


TI = 128
NI = I // TI


def _dense_expert_kernel(x_ref, ws_ref, gp_ref, up_ref, dp_ref, out_ref):
    e = pl.program_id(0)
    i = pl.program_id(1)

    @pl.when((e == 0) & (i == 0))
    def _():
        out_ref[...] = jnp.zeros_like(out_ref)

    x = x_ref[...]
    g = jax.lax.dot_general(
        x, gp_ref[0], (((1,), (1,)), ((), ())), preferred_element_type=jnp.float32
    )
    u = jax.lax.dot_general(
        x, up_ref[0], (((1,), (1,)), ((), ())), preferred_element_type=jnp.float32
    )
    ws_col = jnp.transpose(ws_ref[0])
    inter = g * jax.nn.sigmoid(g) * u * ws_col
    out_ref[...] += jax.lax.dot_general(
        inter, dp_ref[0], (((1,), (1,)), ((), ())), preferred_element_type=jnp.float32
    )


def _dense_experts(x, ws, gate_proj, up_proj, down_proj):
    ws_t = ws.T.reshape(E, 1, T)
    return pl.pallas_call(
        _dense_expert_kernel,
        grid=(E, NI),
        in_specs=[
            pl.BlockSpec((T, H), lambda e, i: (0, 0)),
            pl.BlockSpec((1, 1, T), lambda e, i: (e, 0, 0)),
            pl.BlockSpec((1, TI, H), lambda e, i: (e, i, 0)),
            pl.BlockSpec((1, TI, H), lambda e, i: (e, i, 0)),
            pl.BlockSpec((1, H, TI), lambda e, i: (e, 0, i)),
        ],
        out_specs=pl.BlockSpec((T, H), lambda e, i: (0, 0)),
        out_shape=jax.ShapeDtypeStruct((T, H), jnp.float32),
    )(x, ws_t, gate_proj, up_proj, down_proj)


TS = 256
NS = SI // TS


def _shared_kernel(x_ref, sg_ref, su_ref, sd_ref, out_ref):
    s = pl.program_id(0)

    @pl.when(s == 0)
    def _():
        out_ref[...] = jnp.zeros_like(out_ref)

    x = x_ref[...]
    g = jax.lax.dot_general(
        x, sg_ref[...], (((1,), (1,)), ((), ())), preferred_element_type=jnp.float32
    )
    u = jax.lax.dot_general(
        x, su_ref[...], (((1,), (1,)), ((), ())), preferred_element_type=jnp.float32
    )
    inter = g * jax.nn.sigmoid(g) * u
    out_ref[...] += jax.lax.dot_general(
        inter, sd_ref[...], (((1,), (1,)), ((), ())), preferred_element_type=jnp.float32
    )


def _shared_experts(x, shared_gate_w, shared_up_w, shared_down_w):
    return pl.pallas_call(
        _shared_kernel,
        grid=(NS,),
        in_specs=[
            pl.BlockSpec((T, H), lambda s: (0, 0)),
            pl.BlockSpec((TS, H), lambda s: (s, 0)),
            pl.BlockSpec((TS, H), lambda s: (s, 0)),
            pl.BlockSpec((H, TS), lambda s: (0, s)),
        ],
        out_specs=pl.BlockSpec((T, H), lambda s: (0, 0)),
        out_shape=jax.ShapeDtypeStruct((T, H), jnp.float32),
    )(x, shared_gate_w, shared_up_w, shared_down_w)


def kernel(hidden_states, gate_weight, e_bias, gate_proj, up_proj, down_proj,
           shared_gate_w, shared_up_w, shared_down_w):
    bsz, seq, h = hidden_states.shape
    x = hidden_states.reshape(-1, h)
    ws = _route(x, gate_weight, e_bias)
    y = _dense_experts(x, ws, gate_proj, up_proj, down_proj)
    sh = _shared_experts(x, shared_gate_w, shared_up_w, shared_down_w)
    return (y + sh).reshape(bsz, seq, h)


# speedup vs baseline: 2.2653x; 2.1163x over previous
import functools

import jax
import jax.numpy as jnp
from jax.experimental import pallas as pl
from jax.experimental.pallas import tpu as pltpu

H = 2048
I = 1408
E = 64
TOPK = 8
NG = 8
GS = E // NG
TG = 4
NSH = 2
SI = I * NSH
RSF = 2.5
T = 2048

NEG = -1e30
BT = 128


def _routing_kernel(x_ref, gw_ref, eb_ref, idx_ref, w_ref):
    x = x_ref[...]
    gw = gw_ref[...]
    logits = jax.lax.dot_general(
        x, gw, (((1,), (1,)), ((), ())), preferred_element_type=jnp.float32
    )
    scores = jax.nn.sigmoid(logits)
    sfc = scores + eb_ref[...]

    gs_cols = []
    for g in range(NG):
        sl = sfc[:, g * GS:(g + 1) * GS]
        it = jax.lax.broadcasted_iota(jnp.int32, sl.shape, 1)
        m1 = jnp.max(sl, axis=1, keepdims=True)
        first = jnp.min(jnp.where(sl == m1, it, GS), axis=1, keepdims=True)
        m2 = jnp.max(jnp.where(it == first, NEG, sl), axis=1, keepdims=True)
        gs_cols.append(m1 + m2)
    gsc = jnp.concatenate(gs_cols, axis=1)

    itg = jax.lax.broadcasted_iota(jnp.int32, gsc.shape, 1)
    gmask = jnp.zeros_like(gsc)
    for _ in range(TG):
        m = jnp.max(gsc, axis=1, keepdims=True)
        first = jnp.min(jnp.where(gsc == m, itg, NG), axis=1, keepdims=True)
        sel = itg == first
        gmask = jnp.where(sel, 1.0, gmask)
        gsc = jnp.where(sel, NEG, gsc)
    smask = jnp.concatenate(
        [jnp.broadcast_to(gmask[:, g:g + 1], (gmask.shape[0], GS)) for g in range(NG)],
        axis=1,
    )

    tmp = jnp.where(smask > 0, sfc, 0.0)
    ite = jax.lax.broadcasted_iota(jnp.int32, tmp.shape, 1)
    idx_cols, w_cols = [], []
    denom = jnp.zeros((tmp.shape[0], 1), jnp.float32)
    for _ in range(TOPK):
        m = jnp.max(tmp, axis=1, keepdims=True)
        first = jnp.min(jnp.where(tmp == m, ite, E), axis=1, keepdims=True)
        sel = ite == first
        w = jnp.sum(jnp.where(sel, scores, 0.0), axis=1, keepdims=True)
        idx_cols.append(first)
        w_cols.append(w)
        denom = denom + w
        tmp = jnp.where(sel, NEG, tmp)
    idx_ref[...] = jnp.concatenate(idx_cols, axis=1)
    w_ref[...] = jnp.concatenate(w_cols, axis=1) / (denom + 1e-20) * RSF


def _route(x, gate_weight, e_bias):
    return pl.pallas_call(
        _routing_kernel,
        out_shape=(
            jax.ShapeDtypeStruct((T, TOPK), jnp.int32),
            jax.ShapeDtypeStruct((T, TOPK), jnp.float32),
        ),
    )(x, gate_weight, e_bias.reshape(1, E))


def _dispatch_indices(topk_idx, topk_w):
    A = T * TOPK
    P = A + E * BT
    NB = P // BT
    e_a = topk_idx.reshape(A)
    w_a = topk_w.reshape(A)
    t_a = (jnp.arange(A, dtype=jnp.int32) // TOPK).astype(jnp.int32)
    perm = jnp.argsort(e_a, stable=True)
    es = e_a[perm]
    counts = jnp.sum(
        (e_a[:, None] == jnp.arange(E, dtype=e_a.dtype)[None, :]).astype(jnp.int32),
        axis=0,
    )
    blocks_pe = (counts + BT - 1) // BT
    cumblocks = jnp.cumsum(blocks_pe)
    padded_off = jnp.concatenate(
        [jnp.zeros((1,), jnp.int32), cumblocks[:-1].astype(jnp.int32)]
    ) * BT
    cumcounts = jnp.cumsum(counts)
    unpadded_off = jnp.concatenate(
        [jnp.zeros((1,), jnp.int32), cumcounts[:-1].astype(jnp.int32)]
    )
    rank = jnp.arange(A, dtype=jnp.int32) - unpadded_off[es]
    p = padded_off[es] + rank
    tok_pad = jnp.zeros((P,), jnp.int32).at[p].set(t_a[perm])
    w_pad = jnp.zeros((P,), jnp.float32).at[p].set(w_a[perm])
    block_expert = jnp.minimum(
        jnp.searchsorted(cumblocks, jnp.arange(NB), side="right").astype(jnp.int32),
        E - 1,
    )
    nvalid = cumblocks[-1].astype(jnp.int32).reshape(1)
    pos = jnp.zeros((A,), jnp.int32).at[perm].set(p)
    return tok_pad, w_pad, block_expert, nvalid, pos


def _inter_kernel(be_ref, nv_ref, x_ref, w_ref, gp_ref, up_ref, inter_ref):
    b = pl.program_id(0)

    @pl.when(b < nv_ref[0])
    def _():
        x = x_ref[...]
        g = jax.lax.dot_general(
            x, gp_ref[0], (((1,), (1,)), ((), ())), preferred_element_type=jnp.float32
        )
        u = jax.lax.dot_general(
            x, up_ref[0], (((1,), (1,)), ((), ())), preferred_element_type=jnp.float32
        )
        wcol = jnp.transpose(w_ref[0])
        inter_ref[...] = g * jax.nn.sigmoid(g) * u * wcol


def _down_kernel(be_ref, nv_ref, inter_ref, dp_ref, out_ref):
    b = pl.program_id(0)

    @pl.when(b < nv_ref[0])
    def _():
        out_ref[...] = jax.lax.dot_general(
            inter_ref[...], dp_ref[0], (((1,), (1,)), ((), ())),
            preferred_element_type=jnp.float32,
        )


def _grouped_experts(x_sorted, w_pad, block_expert, nvalid, gate_proj, up_proj,
                     down_proj):
    P = x_sorted.shape[0]
    NB = P // BT
    w3 = w_pad.reshape(NB, 1, BT)
    inter_spec = pltpu.PrefetchScalarGridSpec(
        num_scalar_prefetch=2,
        grid=(NB,),
        in_specs=[
            pl.BlockSpec((BT, H), lambda b, be, nv: (b, 0)),
            pl.BlockSpec((1, 1, BT), lambda b, be, nv: (b, 0, 0)),
            pl.BlockSpec((1, I, H), lambda b, be, nv: (be[b], 0, 0)),
            pl.BlockSpec((1, I, H), lambda b, be, nv: (be[b], 0, 0)),
        ],
        out_specs=pl.BlockSpec((BT, I), lambda b, be, nv: (b, 0)),
    )
    inter = pl.pallas_call(
        _inter_kernel,
        grid_spec=inter_spec,
        out_shape=jax.ShapeDtypeStruct((P, I), jnp.float32),
    )(block_expert, nvalid, x_sorted, w3, gate_proj, up_proj)
    down_spec = pltpu.PrefetchScalarGridSpec(
        num_scalar_prefetch=2,
        grid=(NB,),
        in_specs=[
            pl.BlockSpec((BT, I), lambda b, be, nv: (b, 0)),
            pl.BlockSpec((1, H, I), lambda b, be, nv: (be[b], 0, 0)),
        ],
        out_specs=pl.BlockSpec((BT, H), lambda b, be, nv: (b, 0)),
    )
    return pl.pallas_call(
        _down_kernel,
        grid_spec=down_spec,
        out_shape=jax.ShapeDtypeStruct((P, H), jnp.float32),
    )(block_expert, nvalid, inter, down_proj)


TS = 256


def _shared_kernel(x_ref, sg_ref, su_ref, sd_ref, out_ref):
    s = pl.program_id(0)

    @pl.when(s == 0)
    def _():
        out_ref[...] = jnp.zeros_like(out_ref)

    x = x_ref[...]
    g = jax.lax.dot_general(
        x, sg_ref[...], (((1,), (1,)), ((), ())), preferred_element_type=jnp.float32
    )
    u = jax.lax.dot_general(
        x, su_ref[...], (((1,), (1,)), ((), ())), preferred_element_type=jnp.float32
    )
    inter = g * jax.nn.sigmoid(g) * u
    out_ref[...] += jax.lax.dot_general(
        inter, sd_ref[...], (((1,), (1,)), ((), ())), preferred_element_type=jnp.float32
    )


def _shared_experts(x, shared_gate_w, shared_up_w, shared_down_w):
    ns = SI // TS
    return pl.pallas_call(
        _shared_kernel,
        grid=(ns,),
        in_specs=[
            pl.BlockSpec((T, H), lambda s: (0, 0)),
            pl.BlockSpec((TS, H), lambda s: (s, 0)),
            pl.BlockSpec((TS, H), lambda s: (s, 0)),
            pl.BlockSpec((H, TS), lambda s: (0, s)),
        ],
        out_specs=pl.BlockSpec((T, H), lambda s: (0, 0)),
        out_shape=jax.ShapeDtypeStruct((T, H), jnp.float32),
    )(x, shared_gate_w, shared_up_w, shared_down_w)


def _gather_rows(x, tok_pad):
    return x[tok_pad]


def _combine_rows(out_sorted, pos):
    return out_sorted[pos].reshape(T, TOPK, H).sum(axis=1)


def kernel(hidden_states, gate_weight, e_bias, gate_proj, up_proj, down_proj,
           shared_gate_w, shared_up_w, shared_down_w):
    bsz, seq, h = hidden_states.shape
    x = hidden_states.reshape(-1, h)
    topk_idx, topk_w = _route(x, gate_weight, e_bias)
    tok_pad, w_pad, block_expert, nvalid, pos = _dispatch_indices(topk_idx, topk_w)
    x_sorted = _gather_rows(x, tok_pad)
    out_sorted = _grouped_experts(
        x_sorted, w_pad, block_expert, nvalid, gate_proj, up_proj, down_proj
    )
    y = _combine_rows(out_sorted, pos)
    sh = _shared_experts(x, shared_gate_w, shared_up_w, shared_down_w)
    return (y + sh).reshape(bsz, seq, h)
